# baseline (speedup 1.0000x reference)
import jax
import jax.numpy as jnp
from jax import lax
from jax.experimental import pallas as pl
from jax.experimental.pallas import tpu as pltpu
from jax.experimental.pallas import tpu_sc as plsc

R = 256
C = 24
NC, NS, L = 2, 16, 16
NW = NC * NS
P = 128
NCHUNK = 524288 // NW // P
LSTR = 32


def _coord_setup(v):
    f = (v + 1.0) * ((R - 1) * 0.5)
    i0 = jnp.minimum(f.astype(jnp.int32), R - 2)
    w1 = f - i0.astype(jnp.float32)
    return i0, w1


def _sc_body(xyzc, t0, t1, t2, l_hbm, out_hbm,
             lines_v, xyz_v, idx_v, w_v, ilb_v,
             dA0, dA1, dA2, dA3, dA4, dA5,
             dB0, dB1, dB2, dB3, dB4, dB5,
             out_v, dsem0, dsem1, xsem, osem):
    dsts = ((dA0, dA1, dA2, dA3, dA4, dA5),
            (dB0, dB1, dB2, dB3, dB4, dB5))
    dsems = (dsem0, dsem1)
    tbls = (t0, t1, t2)
    wid = lax.axis_index("s") * NC + lax.axis_index("c")
    cbase = wid * NCHUNK

    pltpu.sync_copy(l_hbm, lines_v)

    def build(k, s):
        for g in range(P // L):
            sl = pl.ds(g * L, L)
            px = xyz_v[s, 0, sl]
            py = xyz_v[s, 1, sl]
            pz = xyz_v[s, 2, sl]
            xi, xw = _coord_setup(px)
            yi, yw = _coord_setup(py)
            zi, zw = _coord_setup(pz)
            r0 = yi * R + xi
            idx_v[s, 0, sl] = r0
            idx_v[s, 1, sl] = r0 + R
            r1 = zi * R + xi
            idx_v[s, 2, sl] = r1
            idx_v[s, 3, sl] = r1 + R
            r2 = zi * R + yi
            idx_v[s, 4, sl] = r2
            idx_v[s, 5, sl] = r2 + R
            w_v[s, 0, sl] = xw
            w_v[s, 1, sl] = xw
            w_v[s, 2, sl] = yw
            w_v[s, 3, sl] = yw
            w_v[s, 4, sl] = zw
            w_v[s, 5, sl] = zw
            w_v[s, 6, sl] = zw
            w_v[s, 7, sl] = yw
            w_v[s, 8, sl] = xw
            ilb_v[s, 0, sl] = zi * LSTR
            ilb_v[s, 1, sl] = yi * LSTR
            ilb_v[s, 2, sl] = xi * LSTR

    def fire(s):
        for p in range(3):
            pltpu.async_copy(tbls[p].at[idx_v.at[s, 2 * p]],
                             dsts[s][2 * p], dsems[s])
            pltpu.async_copy(tbls[p].at[idx_v.at[s, 2 * p + 1]],
                             dsts[s][2 * p + 1], dsems[s])

    def wait_dst(s):
        for p in range(3):
            pltpu.make_async_copy(tbls[p].at[idx_v.at[s, 2 * p]],
                                  dsts[s][2 * p], dsems[s]).wait()
            pltpu.make_async_copy(tbls[p].at[idx_v.at[s, 2 * p + 1]],
                                  dsts[s][2 * p + 1], dsems[s]).wait()

    def compute(k, s):
        @plsc.parallel_loop(0, P // L)
        def _group(g):
            sl = pl.ds(g * L, L)
            ptv = lax.iota(jnp.int32, L) + g * L
            accs = []
            for p in range(3):
                wx1 = w_v[s, p, sl]
                wy1 = w_v[s, 3 + p, sl]
                wl1 = w_v[s, 6 + p, sl]
                il0 = ilb_v[s, p, sl] + p * (R * LSTR)
                d0, d1 = dsts[s][2 * p], dsts[s][2 * p + 1]
                crot = lax.iota(jnp.int32, L)

                def one(c0):
                    c1 = c0 + C
                    v00 = plsc.load_gather(d0, [ptv, c0])
                    v01 = plsc.load_gather(d0, [ptv, c1])
                    v10 = plsc.load_gather(d1, [ptv, c0])
                    v11 = plsc.load_gather(d1, [ptv, c1])
                    la = plsc.load_gather(lines_v, [il0 + c0])
                    lb = plsc.load_gather(lines_v, [il0 + (c0 + LSTR)])
                    t0 = v00 + (v01 - v00) * wx1
                    t1 = v10 + (v11 - v10) * wx1
                    pv = t0 + (t1 - t0) * wy1
                    lv = la + (lb - la) * wl1
                    return pv * lv

                def chan(c, carry):
                    a0, a1 = carry
                    cc = crot + c
                    ca = jnp.where(cc >= C, cc - C, cc)
                    cc2 = cc + C // 2
                    cb = jnp.where(cc2 >= C, cc2 - C, cc2)
                    return (a0 + one(ca), a1 + one(cb))

                zz = jnp.zeros((L,), jnp.float32)
                acc0, acc1 = lax.fori_loop(0, C // 2, chan, (zz, zz),
                                           unroll=4)
                accs.append(acc0 + acc1)
            out_v[s, sl] = accs[0] + accs[1] + accs[2]

    pltpu.sync_copy(xyzc.at[cbase], xyz_v.at[0])
    build(0, 0)
    fire(0)
    pltpu.async_copy(xyzc.at[cbase + 1], xyz_v.at[1], xsem)

    def pair_body(j, _):
        for s in (0, 1):
            k = 2 * j + s
            s2 = 1 - s
            last = (s == 1)

            def stage_next():
                pltpu.make_async_copy(xyzc.at[cbase + k + 1],
                                      xyz_v.at[s2], xsem).wait()
                build(k + 1, s2)
                fire(s2)

            def prefetch_xyz():
                pltpu.async_copy(xyzc.at[cbase + k + 2], xyz_v.at[s], xsem)

            if last:
                pl.when(j < (NCHUNK // 2) - 1)(stage_next)
                pl.when(j < (NCHUNK // 2) - 1)(prefetch_xyz)
            else:
                stage_next()
                pl.when(j < (NCHUNK // 2) - 1)(prefetch_xyz)

            def wait_out_free():
                pltpu.make_async_copy(
                    out_v.at[s], out_hbm.at[pl.ds((cbase + k - 2) * P, P)],
                    osem).wait()

            pl.when(j >= 1)(wait_out_free)
            wait_dst(s)
            compute(k, s)
            pltpu.async_copy(out_v.at[s],
                             out_hbm.at[pl.ds((cbase + k) * P, P)], osem)
        return 0

    lax.fori_loop(0, NCHUNK // 2, pair_body, 0)

    pltpu.make_async_copy(out_v.at[0],
                          out_hbm.at[pl.ds((cbase + NCHUNK - 2) * P, P)],
                          osem).wait()
    pltpu.make_async_copy(out_v.at[1],
                          out_hbm.at[pl.ds((cbase + NCHUNK - 1) * P, P)],
                          osem).wait()


def _make_xpair(plane):
    t = plane[0].transpose(1, 2, 0).reshape(R * R, C)
    return jnp.concatenate([t, jnp.roll(t, -1, axis=0)], axis=1)


@jax.jit
def kernel(in_tensor, plane_yx, line_z, plane_zx, line_y, plane_zy, line_x):
    n = in_tensor.shape[0] * in_tensor.shape[1]
    pts = in_tensor.reshape(n, 3)
    xyzc = pts.reshape(n // P, P, 3).transpose(0, 2, 1)
    t0 = _make_xpair(plane_yx)
    t1 = _make_xpair(plane_zx)
    t2 = _make_xpair(plane_zy)
    def _padline(ln):
        t = ln[0, :, :, 0].T
        return jnp.pad(t, ((0, 0), (0, LSTR - C))).reshape(-1)

    lines = jnp.stack([_padline(line_z), _padline(line_y),
                       _padline(line_x)]).reshape(-1)

    mesh = plsc.VectorSubcoreMesh(core_axis_name="c", subcore_axis_name="s",
                                  num_cores=NC, num_subcores=NS)
    run = pl.kernel(
        _sc_body,
        out_type=jax.ShapeDtypeStruct((n,), jnp.float32),
        mesh=mesh,
        compiler_params=pltpu.CompilerParams(needs_layout_passes=False,
                                             use_tc_tiling_on_sc=False),
        scratch_types=(
            [pltpu.VMEM((3 * R * LSTR,), jnp.float32),
             pltpu.VMEM((2, 3, P), jnp.float32),
             pltpu.VMEM((2, 6, P), jnp.int32),
             pltpu.VMEM((2, 9, P), jnp.float32),
             pltpu.VMEM((2, 3, P), jnp.int32)]
            + [pltpu.VMEM((P, 2 * C), jnp.float32)] * 12
            + [pltpu.VMEM((2, P), jnp.float32),
               pltpu.SemaphoreType.DMA,
               pltpu.SemaphoreType.DMA,
               pltpu.SemaphoreType.DMA,
               pltpu.SemaphoreType.DMA]
        ),
    )
    out = run(xyzc, t0, t1, t2, lines)
    return out.reshape(in_tensor.shape[0], in_tensor.shape[1])


# speedup vs baseline: 1.0077x; 1.0077x over previous
import jax
import jax.numpy as jnp
from jax import lax
from jax.experimental import pallas as pl
from jax.experimental.pallas import tpu as pltpu
from jax.experimental.pallas import tpu_sc as plsc

R = 256
C = 24
NC, NS, L = 2, 16, 16
NW = NC * NS
P = 128
NCHUNK = 524288 // NW // P
LSTR = 32


def _coord_setup(v):
    f = (v + 1.0) * ((R - 1) * 0.5)
    i0 = jnp.minimum(f.astype(jnp.int32), R - 2)
    w1 = f - i0.astype(jnp.float32)
    return i0, w1


def _sc_body(xyzc, t0, t1, t2, l_hbm, out_hbm,
             lines_v, xyz_v, idx_v, w_v, ilb_v,
             dA0, dA1, dA2, dA3, dA4, dA5,
             dB0, dB1, dB2, dB3, dB4, dB5,
             out_v, dsem0, dsem1, xsem, osem):
    dsts = ((dA0, dA1, dA2, dA3, dA4, dA5),
            (dB0, dB1, dB2, dB3, dB4, dB5))
    dsems = (dsem0, dsem1)
    tbls = (t0, t1, t2)
    wid = lax.axis_index("s") * NC + lax.axis_index("c")
    cbase = wid * NCHUNK

    pltpu.sync_copy(l_hbm, lines_v)

    def build(k, s):
        for g in range(P // L):
            sl = pl.ds(g * L, L)
            px = xyz_v[s, 0, sl]
            py = xyz_v[s, 1, sl]
            pz = xyz_v[s, 2, sl]
            xi, xw = _coord_setup(px)
            yi, yw = _coord_setup(py)
            zi, zw = _coord_setup(pz)
            r0 = yi * R + xi
            idx_v[s, 0, sl] = r0
            idx_v[s, 1, sl] = r0 + R
            r1 = zi * R + xi
            idx_v[s, 2, sl] = r1
            idx_v[s, 3, sl] = r1 + R
            r2 = zi * R + yi
            idx_v[s, 4, sl] = r2
            idx_v[s, 5, sl] = r2 + R
            w_v[s, 0, sl] = xw
            w_v[s, 1, sl] = xw
            w_v[s, 2, sl] = yw
            w_v[s, 3, sl] = yw
            w_v[s, 4, sl] = zw
            w_v[s, 5, sl] = zw
            w_v[s, 6, sl] = zw
            w_v[s, 7, sl] = yw
            w_v[s, 8, sl] = xw
            ilb_v[s, 0, sl] = zi * LSTR
            ilb_v[s, 1, sl] = yi * LSTR
            ilb_v[s, 2, sl] = xi * LSTR

    def fire(s):
        for p in range(3):
            pltpu.async_copy(tbls[p].at[idx_v.at[s, 2 * p]],
                             dsts[s][2 * p], dsems[s])
            pltpu.async_copy(tbls[p].at[idx_v.at[s, 2 * p + 1]],
                             dsts[s][2 * p + 1], dsems[s])

    def wait_dst(s):
        for p in range(3):
            pltpu.make_async_copy(tbls[p].at[idx_v.at[s, 2 * p]],
                                  dsts[s][2 * p], dsems[s]).wait()
            pltpu.make_async_copy(tbls[p].at[idx_v.at[s, 2 * p + 1]],
                                  dsts[s][2 * p + 1], dsems[s]).wait()

    def compute(k, s):
        @plsc.parallel_loop(0, P // L)
        def _group(g):
            sl = pl.ds(g * L, L)
            ptv = lax.iota(jnp.int32, L) + g * L
            accs = []
            for p in range(3):
                wx1 = w_v[s, p, sl]
                wy1 = w_v[s, 3 + p, sl]
                wl1 = w_v[s, 6 + p, sl]
                il0 = ilb_v[s, p, sl] + p * (R * LSTR)
                d0, d1 = dsts[s][2 * p], dsts[s][2 * p + 1]
                crot = lax.iota(jnp.int32, L)

                def one(c0):
                    c1 = c0 + C
                    v00 = plsc.load_gather(d0, [ptv, c0])
                    v01 = plsc.load_gather(d0, [ptv, c1])
                    v10 = plsc.load_gather(d1, [ptv, c0])
                    v11 = plsc.load_gather(d1, [ptv, c1])
                    la = plsc.load_gather(lines_v, [il0 + c0])
                    lb = plsc.load_gather(lines_v, [il0 + (c0 + LSTR)])
                    t0 = v00 + (v01 - v00) * wx1
                    t1 = v10 + (v11 - v10) * wx1
                    pv = t0 + (t1 - t0) * wy1
                    lv = la + (lb - la) * wl1
                    return pv * lv

                def chan(c, carry):
                    a0, a1 = carry
                    cc = crot + c
                    ca = jnp.where(cc >= C, cc - C, cc)
                    cc2 = cc + C // 2
                    cb = jnp.where(cc2 >= C, cc2 - C, cc2)
                    return (a0 + one(ca), a1 + one(cb))

                zz = jnp.zeros((L,), jnp.float32)
                acc0, acc1 = lax.fori_loop(0, C // 2, chan, (zz, zz),
                                           unroll=1)
                accs.append(acc0 + acc1)
            out_v[s, sl] = accs[0] + accs[1] + accs[2]

    pltpu.sync_copy(xyzc.at[cbase], xyz_v.at[0])
    build(0, 0)
    fire(0)
    pltpu.async_copy(xyzc.at[cbase + 1], xyz_v.at[1], xsem)

    def pair_body(j, _):
        for s in (0, 1):
            k = 2 * j + s
            s2 = 1 - s
            last = (s == 1)

            def stage_next():
                pltpu.make_async_copy(xyzc.at[cbase + k + 1],
                                      xyz_v.at[s2], xsem).wait()
                build(k + 1, s2)
                fire(s2)

            def prefetch_xyz():
                pltpu.async_copy(xyzc.at[cbase + k + 2], xyz_v.at[s], xsem)

            if last:
                pl.when(j < (NCHUNK // 2) - 1)(stage_next)
                pl.when(j < (NCHUNK // 2) - 1)(prefetch_xyz)
            else:
                stage_next()
                pl.when(j < (NCHUNK // 2) - 1)(prefetch_xyz)

            def wait_out_free():
                pltpu.make_async_copy(
                    out_v.at[s], out_hbm.at[pl.ds((cbase + k - 2) * P, P)],
                    osem).wait()

            pl.when(j >= 1)(wait_out_free)
            wait_dst(s)
            compute(k, s)
            pltpu.async_copy(out_v.at[s],
                             out_hbm.at[pl.ds((cbase + k) * P, P)], osem)
        return 0

    lax.fori_loop(0, NCHUNK // 2, pair_body, 0)

    pltpu.make_async_copy(out_v.at[0],
                          out_hbm.at[pl.ds((cbase + NCHUNK - 2) * P, P)],
                          osem).wait()
    pltpu.make_async_copy(out_v.at[1],
                          out_hbm.at[pl.ds((cbase + NCHUNK - 1) * P, P)],
                          osem).wait()


def _make_xpair(plane):
    t = plane[0].transpose(1, 2, 0).reshape(R * R, C)
    return jnp.concatenate([t, jnp.roll(t, -1, axis=0)], axis=1)


@jax.jit
def kernel(in_tensor, plane_yx, line_z, plane_zx, line_y, plane_zy, line_x):
    n = in_tensor.shape[0] * in_tensor.shape[1]
    pts = in_tensor.reshape(n, 3)
    xyzc = pts.reshape(n // P, P, 3).transpose(0, 2, 1)
    t0 = _make_xpair(plane_yx)
    t1 = _make_xpair(plane_zx)
    t2 = _make_xpair(plane_zy)
    def _padline(ln):
        t = ln[0, :, :, 0].T
        return jnp.pad(t, ((0, 0), (0, LSTR - C))).reshape(-1)

    lines = jnp.stack([_padline(line_z), _padline(line_y),
                       _padline(line_x)]).reshape(-1)

    mesh = plsc.VectorSubcoreMesh(core_axis_name="c", subcore_axis_name="s",
                                  num_cores=NC, num_subcores=NS)
    run = pl.kernel(
        _sc_body,
        out_type=jax.ShapeDtypeStruct((n,), jnp.float32),
        mesh=mesh,
        compiler_params=pltpu.CompilerParams(needs_layout_passes=False,
                                             use_tc_tiling_on_sc=False),
        scratch_types=(
            [pltpu.VMEM((3 * R * LSTR,), jnp.float32),
             pltpu.VMEM((2, 3, P), jnp.float32),
             pltpu.VMEM((2, 6, P), jnp.int32),
             pltpu.VMEM((2, 9, P), jnp.float32),
             pltpu.VMEM((2, 3, P), jnp.int32)]
            + [pltpu.VMEM((P, 2 * C), jnp.float32)] * 12
            + [pltpu.VMEM((2, P), jnp.float32),
               pltpu.SemaphoreType.DMA,
               pltpu.SemaphoreType.DMA,
               pltpu.SemaphoreType.DMA,
               pltpu.SemaphoreType.DMA]
        ),
    )
    out = run(xyzc, t0, t1, t2, lines)
    return out.reshape(in_tensor.shape[0], in_tensor.shape[1])


# speedup vs baseline: 1.0189x; 1.0111x over previous
import jax
import jax.numpy as jnp
from jax import lax
from jax.experimental import pallas as pl
from jax.experimental.pallas import tpu as pltpu
from jax.experimental.pallas import tpu_sc as plsc

R = 256
C = 24
NC, NS, L = 2, 16, 16
NW = NC * NS
P = 128
NCHUNK = 524288 // NW // P
LSTR = 32


def _coord_setup(v):
    f = (v + 1.0) * ((R - 1) * 0.5)
    i0 = jnp.minimum(f.astype(jnp.int32), R - 2)
    w1 = f - i0.astype(jnp.float32)
    return i0, w1


def _sc_body(xyzc, t0, t1, t2, l_hbm, out_hbm,
             lines_v, xyz_v, idx_v, w_v, ilb_v,
             dA0, dA1, dA2, dA3, dA4, dA5,
             dB0, dB1, dB2, dB3, dB4, dB5,
             out_v, dsem0, dsem1, xsem, osem):
    dsts = ((dA0, dA1, dA2, dA3, dA4, dA5),
            (dB0, dB1, dB2, dB3, dB4, dB5))
    dsems = (dsem0, dsem1)
    tbls = (t0, t1, t2)
    wid = lax.axis_index("s") * NC + lax.axis_index("c")
    cbase = wid * NCHUNK

    pltpu.sync_copy(l_hbm, lines_v)

    def build(k, s):
        for g in range(P // L):
            sl = pl.ds(g * L, L)
            px = xyz_v[s, 0, sl]
            py = xyz_v[s, 1, sl]
            pz = xyz_v[s, 2, sl]
            xi, xw = _coord_setup(px)
            yi, yw = _coord_setup(py)
            zi, zw = _coord_setup(pz)
            r0 = yi * R + xi
            idx_v[s, 0, sl] = r0
            idx_v[s, 1, sl] = r0 + R
            r1 = zi * R + xi
            idx_v[s, 2, sl] = r1
            idx_v[s, 3, sl] = r1 + R
            r2 = zi * R + yi
            idx_v[s, 4, sl] = r2
            idx_v[s, 5, sl] = r2 + R
            w_v[s, 0, sl] = xw
            w_v[s, 1, sl] = xw
            w_v[s, 2, sl] = yw
            w_v[s, 3, sl] = yw
            w_v[s, 4, sl] = zw
            w_v[s, 5, sl] = zw
            w_v[s, 6, sl] = zw
            w_v[s, 7, sl] = yw
            w_v[s, 8, sl] = xw
            ilb_v[s, 0, sl] = zi * LSTR
            ilb_v[s, 1, sl] = yi * LSTR
            ilb_v[s, 2, sl] = xi * LSTR

    def fire(s):
        for p in range(3):
            pltpu.async_copy(tbls[p].at[idx_v.at[s, 2 * p]],
                             dsts[s][2 * p], dsems[s])
            pltpu.async_copy(tbls[p].at[idx_v.at[s, 2 * p + 1]],
                             dsts[s][2 * p + 1], dsems[s])

    def wait_dst(s):
        for p in range(3):
            pltpu.make_async_copy(tbls[p].at[idx_v.at[s, 2 * p]],
                                  dsts[s][2 * p], dsems[s]).wait()
            pltpu.make_async_copy(tbls[p].at[idx_v.at[s, 2 * p + 1]],
                                  dsts[s][2 * p + 1], dsems[s]).wait()

    def compute(k, s):
        @plsc.parallel_loop(0, P // L)
        def _group(g):
            sl = pl.ds(g * L, L)
            ptv = lax.iota(jnp.int32, L) + g * L
            accs = []
            for p in range(3):
                wx1 = w_v[s, p, sl]
                wy1 = w_v[s, 3 + p, sl]
                wl1 = w_v[s, 6 + p, sl]
                il0 = ilb_v[s, p, sl] + p * (R * LSTR)
                d0, d1 = dsts[s][2 * p], dsts[s][2 * p + 1]
                crot = lax.iota(jnp.int32, L)

                def one(c0):
                    c1 = c0 + C
                    v00 = plsc.load_gather(d0, [ptv, c0])
                    v01 = plsc.load_gather(d0, [ptv, c1])
                    v10 = plsc.load_gather(d1, [ptv, c0])
                    v11 = plsc.load_gather(d1, [ptv, c1])
                    la = plsc.load_gather(lines_v, [il0 + c0])
                    lb = plsc.load_gather(lines_v, [il0 + (c0 + LSTR)])
                    t0 = v00 + (v01 - v00) * wx1
                    t1 = v10 + (v11 - v10) * wx1
                    pv = t0 + (t1 - t0) * wy1
                    lv = la + (lb - la) * wl1
                    return pv * lv

                def chan(c, carry):
                    a0, a1 = carry
                    cc = crot + c
                    ca = jnp.where(cc >= C, cc - C, cc)
                    cc2 = cc + C // 2
                    cb = jnp.where(cc2 >= C, cc2 - C, cc2)
                    return (a0 + one(ca), a1 + one(cb))

                zz = jnp.zeros((L,), jnp.float32)
                acc0, acc1 = lax.fori_loop(0, C // 2, chan, (zz, zz),
                                           unroll=2)
                accs.append(acc0 + acc1)
            out_v[s, sl] = accs[0] + accs[1] + accs[2]

    pltpu.sync_copy(xyzc.at[cbase], xyz_v.at[0])
    build(0, 0)
    fire(0)
    pltpu.async_copy(xyzc.at[cbase + 1], xyz_v.at[1], xsem)

    def pair_body(j, _):
        for s in (0, 1):
            k = 2 * j + s
            s2 = 1 - s
            last = (s == 1)

            def stage_next():
                pltpu.make_async_copy(xyzc.at[cbase + k + 1],
                                      xyz_v.at[s2], xsem).wait()
                build(k + 1, s2)
                fire(s2)

            def prefetch_xyz():
                pltpu.async_copy(xyzc.at[cbase + k + 2], xyz_v.at[s], xsem)

            if last:
                pl.when(j < (NCHUNK // 2) - 1)(stage_next)
                pl.when(j < (NCHUNK // 2) - 1)(prefetch_xyz)
            else:
                stage_next()
                pl.when(j < (NCHUNK // 2) - 1)(prefetch_xyz)

            def wait_out_free():
                pltpu.make_async_copy(
                    out_v.at[s], out_hbm.at[pl.ds((cbase + k - 2) * P, P)],
                    osem).wait()

            pl.when(j >= 1)(wait_out_free)
            wait_dst(s)
            compute(k, s)
            pltpu.async_copy(out_v.at[s],
                             out_hbm.at[pl.ds((cbase + k) * P, P)], osem)
        return 0

    lax.fori_loop(0, NCHUNK // 2, pair_body, 0)

    pltpu.make_async_copy(out_v.at[0],
                          out_hbm.at[pl.ds((cbase + NCHUNK - 2) * P, P)],
                          osem).wait()
    pltpu.make_async_copy(out_v.at[1],
                          out_hbm.at[pl.ds((cbase + NCHUNK - 1) * P, P)],
                          osem).wait()


def _make_xpair(plane):
    t = plane[0].transpose(1, 2, 0).reshape(R * R, C)
    return jnp.concatenate([t, jnp.roll(t, -1, axis=0)], axis=1)


@jax.jit
def kernel(in_tensor, plane_yx, line_z, plane_zx, line_y, plane_zy, line_x):
    n = in_tensor.shape[0] * in_tensor.shape[1]
    pts = in_tensor.reshape(n, 3)
    xyzc = pts.reshape(n // P, P, 3).transpose(0, 2, 1)
    t0 = _make_xpair(plane_yx)
    t1 = _make_xpair(plane_zx)
    t2 = _make_xpair(plane_zy)
    def _padline(ln):
        t = ln[0, :, :, 0].T
        return jnp.pad(t, ((0, 0), (0, LSTR - C))).reshape(-1)

    lines = jnp.stack([_padline(line_z), _padline(line_y),
                       _padline(line_x)]).reshape(-1)

    mesh = plsc.VectorSubcoreMesh(core_axis_name="c", subcore_axis_name="s",
                                  num_cores=NC, num_subcores=NS)
    run = pl.kernel(
        _sc_body,
        out_type=jax.ShapeDtypeStruct((n,), jnp.float32),
        mesh=mesh,
        compiler_params=pltpu.CompilerParams(needs_layout_passes=False,
                                             use_tc_tiling_on_sc=False),
        scratch_types=(
            [pltpu.VMEM((3 * R * LSTR,), jnp.float32),
             pltpu.VMEM((2, 3, P), jnp.float32),
             pltpu.VMEM((2, 6, P), jnp.int32),
             pltpu.VMEM((2, 9, P), jnp.float32),
             pltpu.VMEM((2, 3, P), jnp.int32)]
            + [pltpu.VMEM((P, 2 * C), jnp.float32)] * 12
            + [pltpu.VMEM((2, P), jnp.float32),
               pltpu.SemaphoreType.DMA,
               pltpu.SemaphoreType.DMA,
               pltpu.SemaphoreType.DMA,
               pltpu.SemaphoreType.DMA]
        ),
    )
    out = run(xyzc, t0, t1, t2, lines)
    return out.reshape(in_tensor.shape[0], in_tensor.shape[1])
